# baseline (speedup 1.0000x reference)
import functools

import jax
import jax.numpy as jnp
from jax import lax
from jax.experimental import pallas as pl
from jax.experimental.pallas import tpu as pltpu
from jax.experimental.pallas import tpu_sc as plsc


@functools.cache
def _make_kernel(NE, NR, D, B):
    info = plsc.get_sparse_core_info()
    NC, NS = info.num_cores, info.num_subcores
    NW = NC * NS
    assert B % (8 * NW) == 0, "batch must split 8-aligned across subcores"
    bpw = B // NW
    mesh = plsc.VectorSubcoreMesh(core_axis_name="c", subcore_axis_name="s")

    f32 = jnp.float32
    out_row = jax.ShapeDtypeStruct((B, D), f32)

    @functools.partial(
        pl.kernel,
        mesh=mesh,
        out_type=(out_row, out_row, out_row),
        compiler_params=pltpu.CompilerParams(use_tc_tiling_on_sc=False),
        scratch_types=[
            pltpu.VMEM((bpw,), jnp.int32),
            pltpu.VMEM((bpw,), jnp.int32),
            pltpu.VMEM((bpw,), jnp.int32),
            pltpu.VMEM((bpw, D), f32),
            pltpu.VMEM((bpw, D), f32),
            pltpu.VMEM((bpw, D), f32),
            pltpu.SemaphoreType.DMA,
            pltpu.SemaphoreType.DMA,
            pltpu.SemaphoreType.DMA,
            pltpu.SemaphoreType.DMA,
            pltpu.SemaphoreType.DMA,
            pltpu.SemaphoreType.DMA,
        ],
    )
    def k(ent_hbm, rel_hbm, h_hbm, t_hbm, r_hbm,
          ho_hbm, to_hbm, ro_hbm,
          hi_v, ti_v, ri_v, hrows_v, trows_v, rrows_v,
          sem_h, sem_t, sem_r, sem_ho, sem_to, sem_ro):
        wid = lax.axis_index("s") * NC + lax.axis_index("c")
        base = wid * bpw
        pltpu.sync_copy(h_hbm.at[pl.ds(base, bpw)], hi_v)
        pltpu.sync_copy(t_hbm.at[pl.ds(base, bpw)], ti_v)
        pltpu.sync_copy(r_hbm.at[pl.ds(base, bpw)], ri_v)
        ch = pltpu.async_copy(ent_hbm.at[hi_v], hrows_v, sem_h)
        ct = pltpu.async_copy(ent_hbm.at[ti_v], trows_v, sem_t)
        cr = pltpu.async_copy(rel_hbm.at[ri_v], rrows_v, sem_r)
        ch.wait()
        co_h = pltpu.async_copy(hrows_v, ho_hbm.at[pl.ds(base, bpw)], sem_ho)
        ct.wait()
        co_t = pltpu.async_copy(trows_v, to_hbm.at[pl.ds(base, bpw)], sem_to)
        cr.wait()
        co_r = pltpu.async_copy(rrows_v, ro_hbm.at[pl.ds(base, bpw)], sem_ro)
        co_h.wait()
        co_t.wait()
        co_r.wait()

    return k


def kernel(entity_table, relation_table, h, t, r):
    B = h.shape[0]
    D = entity_table.shape[1]
    k = _make_kernel(entity_table.shape[0], relation_table.shape[0], D, B)
    return k(entity_table, relation_table,
             h.astype(jnp.int32), t.astype(jnp.int32), r.astype(jnp.int32))


# speedup vs baseline: 1.6595x; 1.6595x over previous
import functools

import jax
import jax.numpy as jnp
from jax import lax
from jax.experimental import pallas as pl
from jax.experimental.pallas import tpu as pltpu
from jax.experimental.pallas import tpu_sc as plsc


@functools.cache
def _make_kernel(NE, NR, D, B):
    info = plsc.get_sparse_core_info()
    NC, NS = info.num_cores, info.num_subcores
    NW = NC * NS
    assert B % (8 * NW) == 0
    bpw = B // NW
    mesh = plsc.VectorSubcoreMesh(core_axis_name="c", subcore_axis_name="s")

    f32 = jnp.float32
    out_row = jax.ShapeDtypeStruct((B, D), f32)

    @functools.partial(
        pl.kernel,
        mesh=mesh,
        out_type=(out_row, out_row, out_row),
        scratch_types=[
            pltpu.VMEM((bpw,), jnp.int32),
            pltpu.VMEM((bpw, D), f32),
            pltpu.SemaphoreType.DMA,
            pltpu.SemaphoreType.DMA,
        ],
    )
    def k(ent_hbm, rel_hbm, h_hbm, t_hbm, r_hbm,
          ho_hbm, to_hbm, ro_hbm,
          idx_v, rows_v, sem, sem_out):
        wid = lax.axis_index("s") * NC + lax.axis_index("c")
        base = wid * bpw

        def gather_one(tab_hbm, i_hbm, o_hbm):
            pltpu.sync_copy(i_hbm.at[pl.ds(base, bpw)], idx_v)

            def body(c, carry):
                ivec = idx_v[pl.ds(c * 16, 16)]
                for j in range(16):
                    idx = ivec[j]
                    pltpu.async_copy(
                        tab_hbm.at[pl.ds(idx, 1), :],
                        rows_v.at[pl.ds(c * 16 + j, 1), :],
                        sem,
                    )
                return carry

            lax.fori_loop(0, bpw // 16, body, 0)
            pltpu.make_async_copy(tab_hbm.at[pl.ds(0, bpw), :], rows_v, sem).wait()
            co = pltpu.async_copy(rows_v, o_hbm.at[pl.ds(base, bpw)], sem_out)
            return co

        c1 = gather_one(ent_hbm, h_hbm, ho_hbm)
        c1.wait()
        c2 = gather_one(ent_hbm, t_hbm, to_hbm)
        c2.wait()
        c3 = gather_one(rel_hbm, r_hbm, ro_hbm)
        c3.wait()

    return k


def kernel(entity_table, relation_table, h, t, r):
    B = h.shape[0]
    D = entity_table.shape[1]
    k = _make_kernel(entity_table.shape[0], relation_table.shape[0], D, B)
    return k(entity_table, relation_table,
             h.astype(jnp.int32), t.astype(jnp.int32), r.astype(jnp.int32))
